# Initial kernel scaffold; baseline (speedup 1.0000x reference)
import jax
import jax.numpy as jnp
from jax.experimental import pallas as pl


def kernel(x, edge_index, edge_attr, u, batch, W1, b1, W2, b2):
    raise NotImplementedError("write your pallas kernel here")


# speedup vs baseline: 3.1788x; 3.1788x over previous
import functools

import jax
import jax.numpy as jnp
from jax import lax
from jax.experimental import pallas as pl
from jax.experimental.pallas import tpu as pltpu
from jax.experimental.pallas import tpu_sc as plsc

N_NODES = 10000
N_EDGES = 320000
D = 128
D_EDGE = 16
N_GRAPHS = 8
U_DIM = 64

NC, NS = 2, 16
NW = NC * NS
CH = 128
NCHUNK = N_EDGES // CH
N_PAD = 10240
RPS = N_PAD // NS

_f32 = jnp.float32


def _y_body(x_ref, w_ref, o_ref):
    o_ref[...] = lax.dot_general(
        x_ref[...], w_ref[...], (((1,), (1,)), ((), ())),
        preferred_element_type=_f32)


def _e_body(ea_ref, w_ref, b_ref, o_ref):
    o_ref[...] = lax.dot_general(
        ea_ref[...], w_ref[...], (((1,), (1,)), ((), ())),
        preferred_element_type=_f32) + b_ref[...]


def _sc_body(src_h, dst_h, e_h, y_h, zeros_h, out_h,
             acc, idxs, idxd, rows, ev, semg, seme):
    c = lax.axis_index("c")
    s = lax.axis_index("s")
    t = c * NS + s

    pltpu.sync_copy(zeros_h, acc.at[pl.ds(s * RPS, RPS)])
    plsc.subcore_barrier()

    lo = t * NCHUNK // NW
    hi = (t + 1) * NCHUNK // NW

    def chunk(j, carry):
        pltpu.sync_copy(src_h.at[pl.ds(j * CH, CH)], idxs)
        pltpu.sync_copy(dst_h.at[pl.ds(j * CH, CH)], idxd)
        cp_e = pltpu.async_copy(e_h.at[j], ev, seme)
        cp_g = pltpu.async_copy(y_h.at[idxs], rows, semg)
        cp_e.wait()
        cp_g.wait()

        def comp(i, carry2):
            for q in range(D // 16):
                sl = pl.ds(q * 16, 16)
                rows[i, sl] = jnp.maximum(rows[i, sl] + ev[i, sl], 0.0)
            return carry2

        lax.fori_loop(0, CH, comp, 0)
        pltpu.sync_copy(rows, acc.at[idxd], add=True)
        return carry

    lax.fori_loop(lo, hi, chunk, 0)
    plsc.subcore_barrier()
    pltpu.sync_copy(acc.at[pl.ds(s * RPS, RPS)],
                    out_h.at[c, pl.ds(s * RPS, RPS)])


@functools.cache
def _sc_call():
    return pl.kernel(
        _sc_body,
        out_type=jax.ShapeDtypeStruct((NC, N_PAD, D), _f32),
        mesh=plsc.VectorSubcoreMesh(core_axis_name="c", subcore_axis_name="s"),
        scratch_types=[
            pltpu.VMEM_SHARED((N_PAD, D), _f32),
            pltpu.VMEM((CH,), jnp.int32),
            pltpu.VMEM((CH,), jnp.int32),
            pltpu.VMEM((CH, D), _f32),
            pltpu.VMEM((CH, D), _f32),
            pltpu.SemaphoreType.DMA,
            pltpu.SemaphoreType.DMA,
        ],
    )


def _node_body(x_ref, p_ref, b3_ref, u_ref, w2_ref, b2_ref, o_ref):
    w2 = w2_ref[...]
    p = p_ref[...]
    agg = p[0] + p[1]
    acc = lax.dot_general(x_ref[...], w2[:, :D], (((1,), (1,)), ((), ())),
                          preferred_element_type=_f32)
    acc += lax.dot_general(agg, w2[:, D:2 * D], (((1,), (1,)), ((), ())),
                           preferred_element_type=_f32)
    uu = lax.dot_general(u_ref[...], w2[:, 2 * D:], (((1,), (1,)), ((), ())),
                         preferred_element_type=_f32)
    b = b3_ref[...].reshape(-1)
    oh = (b[:, None] == lax.broadcasted_iota(jnp.int32, (b.shape[0], N_GRAPHS), 1))
    acc += jnp.dot(oh.astype(_f32), uu, preferred_element_type=_f32)
    o_ref[...] = jnp.maximum(acc + b2_ref[...], 0.0)


def kernel(x, edge_index, edge_attr, u, batch, W1, b1, W2, b2):
    dst = edge_index[0].astype(jnp.int32)
    src = edge_index[1].astype(jnp.int32)

    y = pl.pallas_call(
        _y_body,
        grid=(10,),
        in_specs=[pl.BlockSpec((1000, D), lambda i: (i, 0)),
                  pl.BlockSpec((D, D), lambda i: (0, 0))],
        out_specs=pl.BlockSpec((1000, D), lambda i: (i, 0)),
        out_shape=jax.ShapeDtypeStruct((N_NODES, D), _f32),
    )(x, W1[:, :D])

    EB = 3200
    e = pl.pallas_call(
        _e_body,
        grid=(N_EDGES // EB,),
        in_specs=[pl.BlockSpec((EB, D_EDGE), lambda i: (i, 0)),
                  pl.BlockSpec((D, D_EDGE), lambda i: (0, 0)),
                  pl.BlockSpec((1, D), lambda i: (0, 0))],
        out_specs=pl.BlockSpec((EB, D), lambda i: (i, 0)),
        out_shape=jax.ShapeDtypeStruct((N_EDGES, D), _f32),
    )(edge_attr, W1[:, D:], b1.reshape(1, D))

    parts = _sc_call()(src, dst, e.reshape(NCHUNK, CH, D), y,
                       jnp.zeros((RPS, D), _f32))

    NB = 1000
    out = pl.pallas_call(
        _node_body,
        grid=(N_NODES // NB,),
        in_specs=[pl.BlockSpec((NB, D), lambda i: (i, 0)),
                  pl.BlockSpec((NC, NB, D), lambda i: (0, i, 0)),
                  pl.BlockSpec((1, 1, NB), lambda i: (i, 0, 0)),
                  pl.BlockSpec((N_GRAPHS, U_DIM), lambda i: (0, 0)),
                  pl.BlockSpec((D, 2 * D + U_DIM), lambda i: (0, 0)),
                  pl.BlockSpec((1, D), lambda i: (0, 0))],
        out_specs=pl.BlockSpec((NB, D), lambda i: (i, 0)),
        out_shape=jax.ShapeDtypeStruct((N_NODES, D), _f32),
    )(x, parts, batch.astype(jnp.int32).reshape(N_NODES // NB, 1, NB),
      u, W2, b2.reshape(1, D))

    return out


# speedup vs baseline: 4.6404x; 1.4598x over previous
import functools

import jax
import jax.numpy as jnp
from jax import lax
from jax.experimental import pallas as pl
from jax.experimental.pallas import tpu as pltpu
from jax.experimental.pallas import tpu_sc as plsc

N_NODES = 10000
N_EDGES = 320000
D = 128
D_EDGE = 16
N_GRAPHS = 8
U_DIM = 64

NC, NS = 2, 16
NW = NC * NS
CH = 40
NCHUNK = N_EDGES // CH
CPT = NCHUNK // NW
NBUF = 3
N_PAD = 10112
RPS = N_PAD // NS
DP = D // 2
SCALE = 4096.0
CLIP = 7.995

_f32 = jnp.float32


def _pack(v):
    q = jnp.clip(v, -CLIP, CLIP) * SCALE
    q = jnp.where(q >= 0, q + 0.5, q - 0.5)
    a = q[:, :DP].astype(jnp.int32)
    b = q[:, DP:].astype(jnp.int32)
    return (a & 0xFFFF) | (b << 16)


def _y_body(x_ref, w_ref, o_ref):
    o_ref[...] = lax.dot_general(
        x_ref[...], w_ref[...], (((1,), (1,)), ((), ())),
        preferred_element_type=_f32)


def _e_body(ea_ref, w_ref, b_ref, o_ref):
    o_ref[...] = lax.dot_general(
        ea_ref[...], w_ref[...], (((1,), (1,)), ((), ())),
        preferred_element_type=_f32) + b_ref[...]


def _sc_body(src_h, dst_h, e_h, y_h, zeros_h, out_h,
             rg0, rg1, rg2, ev0, ev1, ev2, rf0, rf1, rf2,
             is0, is1, is2, id0, id1, id2, id3, id4, id5,
             acc,
             semg0, semg1, semg2, seme0, seme1, seme2,
             semsc0, semsc1, semsc2, semis0, semis1, semis2,
             semid0, semid1, semid2, semid3, semid4, semid5):
    c = lax.axis_index("c")
    s = lax.axis_index("s")
    t = c * NS + s
    rg = (rg0, rg1, rg2)
    ev = (ev0, ev1, ev2)
    rf = (rf0, rf1, rf2)
    idxs = (is0, is1, is2)
    idxd = (id0, id1, id2, id3, id4, id5)
    semg = (semg0, semg1, semg2)
    seme = (seme0, seme1, seme2)
    semsc = (semsc0, semsc1, semsc2)
    semis = (semis0, semis1, semis2)
    semid = (semid0, semid1, semid2, semid3, semid4, semid5)

    pltpu.sync_copy(zeros_h, acc.at[pl.ds(s * RPS, RPS)])
    plsc.subcore_barrier()

    base = t * CPT

    def issue_idx_s(jj, p):
        pltpu.async_copy(src_h.at[pl.ds((base + jj) * CH, CH)], idxs[p],
                         semis[p])

    def issue_idx_d(jj, d):
        pltpu.async_copy(dst_h.at[pl.ds((base + jj) * CH, CH)], idxd[d],
                         semid[d])

    def issue_data(jj, p):
        pltpu.async_copy(e_h.at[base + jj], ev[p], seme[p])
        pltpu.async_copy(y_h.at[idxs[p]], rg[p], semg[p])

    def wait_data(jj, p):
        pltpu.make_async_copy(e_h.at[base + jj], ev[p], seme[p]).wait()
        pltpu.make_async_copy(y_h.at[idxs[p]], rg[p], semg[p]).wait()

    def wait_scatter(p):
        pltpu.make_async_copy(rf[p], acc.at[idxd[0]], semsc[p]).wait()

    def compute(p):
        def comp(i, carry2):
            for g in range(D // 16):
                sl = pl.ds(g * 16, 16)
                rf[p][i, sl] = jnp.maximum(rg[p][i, sl] + ev[p][i, sl], 0.0)
            return carry2

        lax.fori_loop(0, CH, comp, 0)

    def phase(jj, p, d6, warm, more):
        wait_data(jj, p)
        if more:
            issue_idx_s(jj + 3, p)
        if warm:
            wait_scatter(p)
        if more:
            issue_idx_d(jj + 3, (d6 + 3) % 6)
        compute(p)
        if warm:
            pltpu.make_async_copy(dst_h.at[pl.ds(base * CH, CH)], idxd[d6],
                                  semid[d6]).wait()
        pltpu.async_copy(rf[p], acc.at[idxd[d6]], semsc[p], add=True)
        if more:
            pltpu.make_async_copy(src_h.at[pl.ds(base * CH, CH)], idxs[p],
                                  semis[p]).wait()
            issue_data(jj + 3, p)

    for j0 in range(NBUF):
        pltpu.sync_copy(src_h.at[pl.ds((base + j0) * CH, CH)], idxs[j0])
        pltpu.sync_copy(dst_h.at[pl.ds((base + j0) * CH, CH)], idxd[j0])
    for j0 in range(NBUF):
        issue_data(j0, j0)
    for j0 in range(NBUF):
        phase(j0, j0, j0, False, True)

    def hexa(i, carry):
        jj = 6 * i + 3
        for k in range(6):
            phase(jj + k, (3 + k) % 3, (3 + k) % 6, True, True)
        return carry

    lax.fori_loop(0, (CPT - NBUF - 7) // 6, hexa, 0)
    for j0 in range(CPT - 7, CPT):
        phase(j0, j0 % 3, j0 % 6, True, j0 + 3 < CPT)

    wait_scatter(1)
    wait_scatter(2)
    wait_scatter(0)

    plsc.subcore_barrier()
    pltpu.sync_copy(acc.at[pl.ds(s * RPS, RPS)],
                    out_h.at[c, pl.ds(s * RPS, RPS)])


@functools.cache
def _sc_call():
    return pl.kernel(
        _sc_body,
        out_type=jax.ShapeDtypeStruct((NC, N_PAD, D), _f32),
        mesh=plsc.VectorSubcoreMesh(core_axis_name="c", subcore_axis_name="s"),
        scratch_types=(
            [pltpu.VMEM((CH, D), _f32) for _ in range(NBUF)] +
            [pltpu.VMEM((CH, D), _f32) for _ in range(NBUF)] +
            [pltpu.VMEM((CH, D), _f32) for _ in range(NBUF)] +
            [pltpu.VMEM((CH,), jnp.int32) for _ in range(3)] +
            [pltpu.VMEM((CH,), jnp.int32) for _ in range(6)] +
            [pltpu.VMEM_SHARED((N_PAD, D), _f32)] +
            [pltpu.SemaphoreType.DMA for _ in range(18)]
        ),
    )


def _node_body(x_ref, p_ref, b3_ref, u_ref, w2_ref, b2_ref, o_ref):
    w2 = w2_ref[...]
    p = p_ref[...]
    agg = p[0] + p[1]
    acc = lax.dot_general(x_ref[...], w2[:, :D], (((1,), (1,)), ((), ())),
                          preferred_element_type=_f32)
    acc += lax.dot_general(agg, w2[:, D:2 * D], (((1,), (1,)), ((), ())),
                           preferred_element_type=_f32)
    uu = lax.dot_general(u_ref[...], w2[:, 2 * D:], (((1,), (1,)), ((), ())),
                         preferred_element_type=_f32)
    b = b3_ref[...].reshape(-1)
    oh = (b[:, None] == lax.broadcasted_iota(jnp.int32, (b.shape[0], N_GRAPHS), 1))
    acc += jnp.dot(oh.astype(_f32), uu, preferred_element_type=_f32)
    o_ref[...] = jnp.maximum(acc + b2_ref[...], 0.0)


def kernel(x, edge_index, edge_attr, u, batch, W1, b1, W2, b2):
    dst = edge_index[0].astype(jnp.int32)
    src = edge_index[1].astype(jnp.int32)

    y = pl.pallas_call(
        _y_body,
        grid=(10,),
        in_specs=[pl.BlockSpec((1000, D), lambda i: (i, 0)),
                  pl.BlockSpec((D, D), lambda i: (0, 0))],
        out_specs=pl.BlockSpec((1000, D), lambda i: (i, 0)),
        out_shape=jax.ShapeDtypeStruct((N_NODES, D), _f32),
    )(x, W1[:, :D])

    EB = 3200
    e = pl.pallas_call(
        _e_body,
        grid=(N_EDGES // EB,),
        in_specs=[pl.BlockSpec((EB, D_EDGE), lambda i: (i, 0)),
                  pl.BlockSpec((D, D_EDGE), lambda i: (0, 0)),
                  pl.BlockSpec((1, D), lambda i: (0, 0))],
        out_specs=pl.BlockSpec((EB, D), lambda i: (i, 0)),
        out_shape=jax.ShapeDtypeStruct((N_EDGES, D), _f32),
    )(edge_attr, W1[:, D:], b1.reshape(1, D))

    parts = _sc_call()(src, dst, e.reshape(NCHUNK, CH, D), y,
                       jnp.zeros((RPS, D), _f32))

    NB = 1000
    out = pl.pallas_call(
        _node_body,
        grid=(N_NODES // NB,),
        in_specs=[pl.BlockSpec((NB, D), lambda i: (i, 0)),
                  pl.BlockSpec((NC, NB, D), lambda i: (0, i, 0)),
                  pl.BlockSpec((1, 1, NB), lambda i: (i, 0, 0)),
                  pl.BlockSpec((N_GRAPHS, U_DIM), lambda i: (0, 0)),
                  pl.BlockSpec((D, 2 * D + U_DIM), lambda i: (0, 0)),
                  pl.BlockSpec((1, D), lambda i: (0, 0))],
        out_specs=pl.BlockSpec((NB, D), lambda i: (i, 0)),
        out_shape=jax.ShapeDtypeStruct((N_NODES, D), _f32),
    )(x, parts, batch.astype(jnp.int32).reshape(N_NODES // NB, 1, NB),
      u, W2, b2.reshape(1, D))

    return out
